# baseline (speedup 1.0000x reference)
import functools

import jax
import jax.numpy as jnp
from jax import lax
from jax.experimental import pallas as pl
from jax.experimental.pallas import tpu as pltpu
from jax.experimental.pallas import tpu_sc as plsc

DIM = 1024
L = 16
NV = DIM // L
NC, NS = 2, 16
NW = NC * NS
RP = 16
R = 2 * RP
EPS = 1e-12


def _sc_gather_pack_kernel(n_tokens, ids_hbm, word_hbm, out_hbm,
                           idx_v, fbuf, obuf, gsem, wsem):
    wid = lax.axis_index("s") * NC + lax.axis_index("c")
    per_w = n_tokens // NW
    half = per_w // 2
    base = wid * per_w
    n_chunks = half // RP

    pltpu.sync_copy(ids_hbm.at[pl.ds(base, per_w)], idx_v)

    def start_gather(k, b):
        return pltpu.async_copy(
            word_hbm.at[idx_v.at[pl.ds(k * R, R)]], fbuf.at[b], gsem)

    start_gather(0, 0)

    def chunk_body(k, _):
        b = k % 2

        @pl.when(k + 1 < n_chunks)
        def _():
            start_gather(k + 1, (k + 1) % 2)

        pltpu.make_async_copy(
            word_hbm.at[idx_v.at[pl.ds(k * R, R)]], fbuf.at[b], gsem).wait()

        @pl.when(k >= 2)
        def _():
            pltpu.make_async_copy(
                obuf.at[b], out_hbm.at[pl.ds(0, RP)], wsem).wait()

        def pack_body(t, _):
            for j in range(NV):
                sl = pl.ds(j * L, L)
                a = fbuf[b, t, sl]
                bb = fbuf[b, t + RP, sl]
                p = plsc.pack(a, bb, format=plsc.PackFormat.INTERLEAVED)
                obuf[b, t, sl] = plsc.bitcast(p, jnp.int32)
            return 0

        lax.fori_loop(0, RP, pack_body, 0)
        pltpu.async_copy(
            obuf.at[b], out_hbm.at[pl.ds(wid * half + k * RP, RP)], wsem)
        return 0

    lax.fori_loop(0, n_chunks, chunk_body, 0)
    for _ in range(2):
        pltpu.make_async_copy(
            obuf.at[0], out_hbm.at[pl.ds(0, RP)], wsem).wait()


def _tc_ln_kernel(x_ref, mfa_ref, mfb_ref, t0_ref, t1_ref, p0_ref, p1_ref,
                  gam_ref, bet_ref, o_ref):
    c0 = t0_ref[...] + p0_ref[...]
    cd = t1_ref[...] + p1_ref[...] - c0
    gam = gam_ref[...]
    bet = bet_ref[...]
    w = x_ref[...]
    half = w.shape[0]
    ya = lax.bitcast_convert_type(w << 16, jnp.float32)
    yb = lax.bitcast_convert_type(
        w & jnp.int32(-65536), jnp.float32)
    for h, (y, mf_ref) in enumerate(((ya, mfa_ref), (yb, mfb_ref))):
        yy = y + c0 + mf_ref[...] * cd
        mean = jnp.mean(yy, axis=1, keepdims=True)
        var = jnp.mean(yy * yy, axis=1, keepdims=True) - mean * mean
        r = lax.rsqrt(var + EPS)
        o_ref[pl.ds(h * half, half)] = (yy - mean) * r * gam + bet


def kernel(input_ids, attention_mask, token_type_ids, word_emb, pos_emb,
           type_emb, ln_gamma, ln_beta):
    b, s = input_ids.shape
    n = b * s
    nh = n // 2
    per_w = n // NW
    half = per_w // 2
    ids = input_ids.reshape(n).astype(jnp.int32)
    ids_p = ids.reshape(NW, 2, half // RP, RP).transpose(0, 2, 1, 3)
    ids_p = ids_p.reshape(n)
    maskf = attention_mask.reshape(NW, 2, half).astype(jnp.float32)
    mfa = maskf[:, 0].reshape(nh, 1)
    mfb = maskf[:, 1].reshape(nh, 1)

    mesh = plsc.VectorSubcoreMesh(
        core_axis_name="c", subcore_axis_name="s",
        num_cores=NC, num_subcores=NS)
    gather_f = pl.kernel(
        functools.partial(_sc_gather_pack_kernel, n),
        out_type=jax.ShapeDtypeStruct((nh, DIM), jnp.int32),
        mesh=mesh,
        compiler_params=pltpu.CompilerParams(needs_layout_passes=False),
        scratch_types=[
            pltpu.VMEM((per_w,), jnp.int32),
            pltpu.VMEM((2, R, DIM), jnp.float32),
            pltpu.VMEM((2, RP, DIM), jnp.int32),
            pltpu.SemaphoreType.DMA,
            pltpu.SemaphoreType.DMA,
        ],
    )
    packed = gather_f(ids_p, word_emb)

    fixed = lambda i: (0, 0)
    row = lambda i: (i, 0)
    out = pl.pallas_call(
        _tc_ln_kernel,
        grid=(NW,),
        in_specs=[
            pl.BlockSpec((half, DIM), row),
            pl.BlockSpec((half, 1), row),
            pl.BlockSpec((half, 1), row),
            pl.BlockSpec((1, DIM), fixed),
            pl.BlockSpec((1, DIM), fixed),
            pl.BlockSpec((1, DIM), fixed),
            pl.BlockSpec((1, DIM), fixed),
            pl.BlockSpec((1, DIM), fixed),
            pl.BlockSpec((1, DIM), fixed),
        ],
        out_specs=pl.BlockSpec((per_w, DIM), row),
        out_shape=jax.ShapeDtypeStruct((n, DIM), jnp.float32),
        compiler_params=pltpu.CompilerParams(
            dimension_semantics=("parallel",)),
    )(packed, mfa, mfb, type_emb[0:1], type_emb[1:2], pos_emb[0:1],
      pos_emb[1:2], ln_gamma[None, :], ln_beta[None, :])
    return out.reshape(b, s, DIM)


# speedup vs baseline: 1.3241x; 1.3241x over previous
import functools

import jax
import jax.numpy as jnp
from jax import lax
from jax.experimental import pallas as pl
from jax.experimental.pallas import tpu as pltpu
from jax.experimental.pallas import tpu_sc as plsc

DIM = 1024
NC, NS = 2, 16
NW = NC * NS
R = 32
NB = 3
BT = 2048
EPS = 1e-12


def _sc_gather_kernel(n_tokens, ids_hbm, word_hbm, out_hbm, idx_v, buf,
                      gsem, wsem):
    wid = lax.axis_index("s") * NC + lax.axis_index("c")
    per_w = n_tokens // NW
    base = wid * per_w
    n_chunks = per_w // R

    pltpu.sync_copy(ids_hbm.at[pl.ds(base, per_w)], idx_v)

    def start_gather(k, b):
        return pltpu.async_copy(
            word_hbm.at[idx_v.at[pl.ds(k * R, R)]], buf.at[b], gsem)

    start_gather(0, 0)

    def chunk_body(k, _):
        b = k % NB
        @pl.when(k >= NB - 1)
        def _():
            pltpu.make_async_copy(
                buf.at[(k + 1) % NB], out_hbm.at[pl.ds(0, R)], wsem).wait()

        @pl.when(k + 1 < n_chunks)
        def _():
            start_gather(k + 1, (k + 1) % NB)

        pltpu.make_async_copy(
            word_hbm.at[idx_v.at[pl.ds(k * R, R)]], buf.at[b], gsem).wait()
        pltpu.async_copy(buf.at[b], out_hbm.at[pl.ds(base + k * R, R)], wsem)
        return 0

    lax.fori_loop(0, n_chunks, chunk_body, 0)
    for _ in range(NB - 1):
        pltpu.make_async_copy(
            buf.at[0], out_hbm.at[pl.ds(0, R)], wsem).wait()


def _tc_ln_kernel(x_ref, mf_ref, t0_ref, t1_ref, p0_ref, p1_ref,
                  gam_ref, bet_ref, o_ref):
    c0 = t0_ref[...] + p0_ref[...]
    cd = t1_ref[...] + p1_ref[...] - c0
    y = x_ref[...] + c0 + mf_ref[...] * cd
    mean = jnp.mean(y, axis=1, keepdims=True)
    var = jnp.mean(y * y, axis=1, keepdims=True) - mean * mean
    r = lax.rsqrt(var + EPS)
    o_ref[...] = (y - mean) * r * gam_ref[...] + bet_ref[...]


def kernel(input_ids, attention_mask, token_type_ids, word_emb, pos_emb,
           type_emb, ln_gamma, ln_beta):
    b, s = input_ids.shape
    n = b * s
    ids = input_ids.reshape(n).astype(jnp.int32)
    maskf = attention_mask.reshape(n, 1).astype(jnp.float32)

    mesh = plsc.VectorSubcoreMesh(
        core_axis_name="c", subcore_axis_name="s",
        num_cores=NC, num_subcores=NS)
    gather_f = pl.kernel(
        functools.partial(_sc_gather_kernel, n),
        out_type=jax.ShapeDtypeStruct((n, DIM), jnp.float32),
        mesh=mesh,
        compiler_params=pltpu.CompilerParams(needs_layout_passes=False),
        scratch_types=[
            pltpu.VMEM((n // NW,), jnp.int32),
            pltpu.VMEM((NB, R, DIM), jnp.float32),
            pltpu.SemaphoreType.DMA,
            pltpu.SemaphoreType.DMA,
        ],
    )
    gathered = gather_f(ids, word_emb)

    row = lambda i: (i, 0)
    fixed = lambda i: (0, 0)
    out = pl.pallas_call(
        _tc_ln_kernel,
        grid=(n // BT,),
        in_specs=[
            pl.BlockSpec((BT, DIM), row),
            pl.BlockSpec((BT, 1), row),
            pl.BlockSpec((1, DIM), fixed),
            pl.BlockSpec((1, DIM), fixed),
            pl.BlockSpec((1, DIM), fixed),
            pl.BlockSpec((1, DIM), fixed),
            pl.BlockSpec((1, DIM), fixed),
            pl.BlockSpec((1, DIM), fixed),
        ],
        out_specs=pl.BlockSpec((BT, DIM), row),
        out_shape=jax.ShapeDtypeStruct((n, DIM), jnp.float32),
        compiler_params=pltpu.CompilerParams(
            dimension_semantics=("parallel",)),
    )(gathered, maskf, type_emb[0:1], type_emb[1:2], pos_emb[0:1],
      pos_emb[1:2], ln_gamma[None, :], ln_beta[None, :])
    return out.reshape(b, s, DIM)
